# baseline (speedup 1.0000x reference)
import dataclasses
import functools

import jax
import jax.numpy as jnp
from jax import lax
from jax.experimental import pallas as pl
from jax.experimental.pallas import tpu as pltpu
from jax.experimental.pallas import tpu_sc as plsc

N_NODES = 10000
N_EDGES = 160000
D_IN = 256
D_OUT = 256
DH = 128
NC = 2
NS = 16
CH = 112
NCH = 93
EPT = NCH * CH
E_PAD = NS * EPT
RPT = 624
RTAIL = N_NODES - NS * RPT

_MM_BLOCK = 1000

_BCAST_DNUMS = lax.GatherDimensionNumbers(
    offset_dims=(), collapsed_slice_dims=(0,), start_index_map=(0,)
)


def _lane_broadcast(vec16, j):
    return lax.gather(
        vec16,
        jnp.full((16, 1), j, jnp.int32),
        _BCAST_DNUMS,
        (1,),
        mode=lax.GatherScatterMode.PROMISE_IN_BOUNDS,
    )


def _matmul(x, W, bias2d):
    def body(x_ref, w_ref, b_ref, o_ref):
        o_ref[...] = (
            jnp.dot(x_ref[...], w_ref[...], preferred_element_type=jnp.float32)
            + b_ref[...]
        )

    return pl.pallas_call(
        body,
        grid=(N_NODES // _MM_BLOCK,),
        in_specs=[
            pl.BlockSpec((_MM_BLOCK, D_IN), lambda i: (i, 0)),
            pl.BlockSpec((D_IN, D_OUT), lambda i: (0, 0)),
            pl.BlockSpec((1, D_OUT), lambda i: (0, 0)),
        ],
        out_specs=pl.BlockSpec((_MM_BLOCK, D_OUT), lambda i: (i, 0)),
        out_shape=jax.ShapeDtypeStruct((N_NODES, D_OUT), jnp.float32),
    )(x, W, bias2d)


def _make_sc_agg():
    mesh = plsc.VectorSubcoreMesh(core_axis_name="c", subcore_axis_name="s")

    cp = pltpu.CompilerParams()
    if "needs_layout_passes" in pltpu.CompilerParams.__dataclass_fields__:
        cp = dataclasses.replace(cp, needs_layout_passes=False)

    @functools.partial(
        pl.kernel,
        compiler_params=cp,
        out_type=jax.ShapeDtypeStruct((NC, N_NODES, DH), jnp.float32),
        mesh=mesh,
        scratch_types=[
            pltpu.VMEM((3 * CH, DH), jnp.float32),
            pltpu.VMEM((3, CH), jnp.int32),
            pltpu.VMEM((3, CH), jnp.int32),
            pltpu.VMEM((3, 128), jnp.float32),
            pltpu.VMEM_SHARED((N_NODES, DH), jnp.float32),
        ] + [pltpu.SemaphoreType.DMA] * 15,
    )
    def agg(h2_hbm, sidx_hbm, dix_hbm, w_hbm, out_hbm,
            rows_v, six_v, dix_v, wv, acc_sh, *sems):
        c = lax.axis_index("c")
        s = lax.axis_index("s")
        sem_g = sems[0:3]
        sem_w = sems[3:6]
        sem_si = sems[6:9]
        sem_di = sems[9:12]
        sem_wt = sems[12:15]
        ebase = s * NCH

        zbuf = rows_v.at[pl.ds(0, CH)]

        @pl.loop(0, CH)
        def _(i):
            for r in range(DH // 16):
                zbuf[i, pl.ds(r * 16, 16)] = jnp.zeros((16,), jnp.float32)

        zbase = s * RPT
        for t in range(RPT // CH):
            pltpu.sync_copy(zbuf, acc_sh.at[pl.ds(zbase + t * CH, CH)])
        rem = RPT % CH
        if rem:
            pltpu.sync_copy(
                zbuf.at[pl.ds(0, rem)],
                acc_sh.at[pl.ds(zbase + (RPT // CH) * CH, rem)],
            )

        @pl.when(s == NS - 1)
        def _():
            pltpu.sync_copy(
                zbuf.at[pl.ds(0, RTAIL)],
                acc_sh.at[pl.ds(NS * RPT, RTAIL)],
            )

        plsc.subcore_barrier()

        def start_six(g, b):
            pltpu.async_copy(sidx_hbm.at[c * (NS * NCH) + ebase + g],
                             six_v.at[b], sem_si[b])

        def wait_six(g, b):
            pltpu.make_async_copy(
                sidx_hbm.at[c * (NS * NCH) + ebase + g], six_v.at[b],
                sem_si[b]).wait()

        def start_dix(g, b):
            pltpu.async_copy(dix_hbm.at[ebase + g], dix_v.at[b], sem_di[b])

        def wait_dix(g, b):
            pltpu.make_async_copy(
                dix_hbm.at[ebase + g], dix_v.at[b], sem_di[b]).wait()

        def start_w(g, b):
            pltpu.async_copy(w_hbm.at[ebase + g], wv.at[b], sem_wt[b])

        def wait_w(g, b):
            pltpu.make_async_copy(
                w_hbm.at[ebase + g], wv.at[b], sem_wt[b]).wait()

        def start_gather(b):
            pltpu.async_copy(h2_hbm.at[six_v.at[b]],
                             rows_v.at[pl.ds(b * CH, CH)], sem_g[b])

        def wait_gather(b):
            pltpu.make_async_copy(
                h2_hbm.at[six_v.at[b]], rows_v.at[pl.ds(b * CH, CH)],
                sem_g[b]
            ).wait()

        def start_scatter(b):
            pltpu.async_copy(rows_v.at[pl.ds(b * CH, CH)],
                             acc_sh.at[dix_v.at[b]], sem_w[b], add=True)

        def wait_scatter(b):
            pltpu.make_async_copy(
                rows_v.at[pl.ds(b * CH, CH)], acc_sh.at[dix_v.at[b]],
                sem_w[b]
            ).wait()

        def scale(b):
            @pl.loop(0, CH // 16)
            def _(q):
                w16 = wv[b, pl.ds(q * 16, 16)]
                for j in range(16):
                    wj = _lane_broadcast(w16, j)
                    e = b * CH + q * 16 + j
                    for r in range(DH // 16):
                        sl = pl.ds(r * 16, 16)
                        rows_v[e, sl] = rows_v[e, sl] * wj

        for b in range(3):
            start_six(b, b)
            start_w(b, b)
        start_dix(0, 0)
        start_dix(1, 1)
        wait_six(0, 0)
        start_gather(0)
        wait_six(1, 1)
        start_gather(1)

        @pl.loop(0, NCH // 3)
        def _(k):
            for b in range(3):
                g = 3 * k + b
                b2 = (b + 2) % 3
                wait_gather(b)
                wait_w(g, b)
                scale(b)
                wait_dix(g, b)
                start_scatter(b)

                @pl.when(g + 2 < NCH)
                def _():
                    wait_six(g + 2, b2)

                    @pl.when(g >= 1)
                    def _():
                        wait_scatter(b2)

                    start_gather(b2)
                    start_dix(g + 2, b2)

                @pl.when(g + 3 < NCH)
                def _():
                    start_six(g + 3, b)
                    start_w(g + 3, b)

        wait_scatter((NCH - 3) % 3)
        wait_scatter((NCH - 2) % 3)
        wait_scatter((NCH - 1) % 3)
        plsc.subcore_barrier()
        pltpu.sync_copy(
            acc_sh.at[pl.ds(s * RPT, RPT)],
            out_hbm.at[c].at[pl.ds(s * RPT, RPT)],
        )

        @pl.when(s == NS - 1)
        def _():
            pltpu.sync_copy(
                acc_sh.at[pl.ds(NS * RPT, RTAIL)],
                out_hbm.at[c].at[pl.ds(NS * RPT, RTAIL)],
            )

    return agg


_sc_agg = _make_sc_agg()


def kernel(x, edge_index, edge_weight, W, bias):
    h = _matmul(x, W, bias.reshape(1, D_OUT))
    h2 = h.reshape(2 * N_NODES, DH)

    dst = edge_index[0].astype(jnp.int32)
    src = edge_index[1].astype(jnp.int32)
    w = edge_weight.astype(jnp.float32)

    pad = E_PAD - N_EDGES
    src = jnp.pad(src, (0, pad))
    dst = jnp.pad(dst, (0, pad))
    w = jnp.pad(w, (0, pad))

    sidx = jnp.stack([2 * src, 2 * src + 1]).reshape(2 * NS * NCH, CH)
    dix = dst.reshape(NS * NCH, CH)
    wpad = jnp.pad(w.reshape(NS * NCH, CH), ((0, 0), (0, 128 - CH)))

    out2 = _sc_agg(h2, sidx, dix, wpad)
    return jnp.concatenate([out2[0], out2[1]], axis=1)


# speedup vs baseline: 1.5972x; 1.5972x over previous
import functools

import jax
import jax.numpy as jnp
from jax import lax
from jax.experimental import pallas as pl
from jax.experimental.pallas import tpu as pltpu
from jax.experimental.pallas import tpu_sc as plsc

N_NODES = 10000
N_EDGES = 160000
D_IN = 256
D_OUT = 256
DH = 128
NC = 2
NS = 16
CH = 128
NCH = 79
EPT = NCH * CH
E_PAD = NS * EPT
RPT = 624
RTAIL = N_NODES - NS * RPT

_MM_BLOCK = 1000

_BCAST_DNUMS = lax.GatherDimensionNumbers(
    offset_dims=(), collapsed_slice_dims=(0,), start_index_map=(0,)
)


def _lane_broadcast(vec16, j):
    return lax.gather(
        vec16,
        jnp.full((16, 1), j, jnp.int32),
        _BCAST_DNUMS,
        (1,),
        mode=lax.GatherScatterMode.PROMISE_IN_BOUNDS,
    )


def _matmul(x, W, bias2d):
    def body(x_ref, w_ref, b_ref, o_ref):
        o_ref[...] = (
            jnp.dot(x_ref[...], w_ref[...], preferred_element_type=jnp.float32)
            + b_ref[...]
        )

    return pl.pallas_call(
        body,
        grid=(N_NODES // _MM_BLOCK,),
        in_specs=[
            pl.BlockSpec((_MM_BLOCK, D_IN), lambda i: (i, 0)),
            pl.BlockSpec((D_IN, D_OUT), lambda i: (0, 0)),
            pl.BlockSpec((1, D_OUT), lambda i: (0, 0)),
        ],
        out_specs=pl.BlockSpec((_MM_BLOCK, D_OUT), lambda i: (i, 0)),
        out_shape=jax.ShapeDtypeStruct((N_NODES, D_OUT), jnp.float32),
    )(x, W, bias2d)


def _make_sc_agg():
    mesh = plsc.VectorSubcoreMesh(core_axis_name="c", subcore_axis_name="s")

    @functools.partial(
        pl.kernel,
        out_type=jax.ShapeDtypeStruct((NC, N_NODES, DH), jnp.float32),
        mesh=mesh,
        scratch_types=[
            pltpu.VMEM((NCH, CH), jnp.int32),
            pltpu.VMEM((NCH, CH), jnp.int32),
            pltpu.VMEM((CH, DH), jnp.float32),
            pltpu.VMEM((NCH, CH), jnp.float32),
            pltpu.VMEM_SHARED((N_NODES, DH), jnp.float32),
        ] + [pltpu.SemaphoreType.DMA] * 4,
    )
    def agg(h2_hbm, srcsel_hbm, dst_hbm, w_hbm, out_hbm,
            src_v, dst_v, rows_v, w_v, acc_sh, sg0, sg1, sw0, sw1):
        c = lax.axis_index("c")
        s = lax.axis_index("s")

        @pl.loop(0, CH)
        def _(i):
            for r in range(DH // 16):
                rows_v[i, pl.ds(r * 16, 16)] = jnp.zeros((16,), jnp.float32)

        zbase = s * RPT
        for t in range(RPT // CH):
            pltpu.sync_copy(rows_v, acc_sh.at[pl.ds(zbase + t * CH, CH)])
        rem = RPT % CH
        if rem:
            pltpu.sync_copy(
                rows_v.at[pl.ds(0, rem)],
                acc_sh.at[pl.ds(zbase + (RPT // CH) * CH, rem)],
            )

        @pl.when(s == NS - 1)
        def _():
            pltpu.sync_copy(
                rows_v.at[pl.ds(0, RTAIL)],
                acc_sh.at[pl.ds(NS * RPT, RTAIL)],
            )

        plsc.subcore_barrier()

        pltpu.sync_copy(srcsel_hbm.at[c * NS + s], src_v)
        pltpu.sync_copy(dst_hbm.at[s], dst_v)
        pltpu.sync_copy(w_hbm.at[s], w_v)

        sem_g = (sg0, sg1)
        sem_w = (sw0, sw1)

        def reg(h):
            return rows_v.at[pl.ds(h * 64, 64)]

        def idx(ref, i):
            return ref.at[i // 2, pl.ds((i % 2) * 64, 64)]

        def start_gather(i, h):
            pltpu.async_copy(h2_hbm.at[idx(src_v, i)], reg(h), sem_g[h])

        def wait_gather(i, h):
            pltpu.make_async_copy(h2_hbm.at[idx(src_v, i)], reg(h),
                                  sem_g[h]).wait()

        def start_scatter(i, h):
            pltpu.async_copy(reg(h), acc_sh.at[idx(dst_v, i)], sem_w[h],
                             add=True)

        def wait_scatter(i, h):
            pltpu.make_async_copy(reg(h), acc_sh.at[idx(dst_v, i)],
                                  sem_w[h]).wait()

        def scale(g, h):
            @pl.loop(0, 4)
            def _(q):
                w16 = w_v[g, pl.ds(h * 64 + q * 16, 16)]
                for j in range(16):
                    wj = _lane_broadcast(w16, j)
                    e = h * 64 + q * 16 + j
                    for r in range(DH // 16):
                        sl = pl.ds(r * 16, 16)
                        rows_v[e, sl] = rows_v[e, sl] * wj

        start_gather(0, 0)

        @pl.loop(0, NCH)
        def _(g):
            i0 = 2 * g

            wait_gather(i0, 0)

            @pl.when(g >= 1)
            def _():
                wait_scatter(i0 - 1, 1)

            start_gather(i0 + 1, 1)
            scale(g, 0)
            start_scatter(i0, 0)

            wait_gather(i0 + 1, 1)
            wait_scatter(i0, 0)

            @pl.when(g < NCH - 1)
            def _():
                start_gather(i0 + 2, 0)

            scale(g, 1)
            start_scatter(i0 + 1, 1)

        wait_scatter(2 * NCH - 1, 1)
        plsc.subcore_barrier()
        pltpu.sync_copy(
            acc_sh.at[pl.ds(s * RPT, RPT)],
            out_hbm.at[c].at[pl.ds(s * RPT, RPT)],
        )

        @pl.when(s == NS - 1)
        def _():
            pltpu.sync_copy(
                acc_sh.at[pl.ds(NS * RPT, RTAIL)],
                out_hbm.at[c].at[pl.ds(NS * RPT, RTAIL)],
            )

    return agg


_sc_agg = _make_sc_agg()


def kernel(x, edge_index, edge_weight, W, bias):
    h = _matmul(x, W, bias.reshape(1, D_OUT))
    h2 = h.reshape(2 * N_NODES, DH)

    dst = edge_index[0].astype(jnp.int32)
    src = edge_index[1].astype(jnp.int32)
    w = edge_weight.astype(jnp.float32)

    pad = E_PAD - N_EDGES
    src = jnp.pad(src, (0, pad))
    dst = jnp.pad(dst, (0, pad))
    w = jnp.pad(w, (0, pad))

    srcsel = jnp.stack([2 * src, 2 * src + 1]).reshape(2 * NS, NCH, CH)
    dstr = dst.reshape(NS, NCH, CH)
    wr = w.reshape(NS, NCH, CH)

    out2 = _sc_agg(h2, srcsel, dstr, wr)
    return jnp.concatenate([out2[0], out2[1]], axis=1)
